# Initial kernel scaffold; baseline (speedup 1.0000x reference)
import functools

import jax
import jax.numpy as jnp
from jax import lax
from jax.experimental import pallas as pl
from jax.experimental.pallas import tpu as pltpu
from jax.experimental.pallas import tpu_sc as plsc

_INFO = plsc.get_sparse_core_info()
_NC = _INFO.num_cores
_NS = _INFO.num_subcores
_NW = _NC * _NS
_L = 128
_R = 4


def _gather_impl(n_idx_rows, d, table, idx2d):
    rows_w = n_idx_rows // _NW
    nchunk = rows_w // _R
    chunk = _R * _L
    assert n_idx_rows % _NW == 0 and rows_w % _R == 0
    assert nchunk >= 2 and nchunk % 2 == 0

    mesh = plsc.VectorSubcoreMesh(core_axis_name="c", subcore_axis_name="s")

    @functools.partial(
        pl.kernel,
        mesh=mesh,
        out_type=jax.ShapeDtypeStruct((n_idx_rows * _L, d), jnp.float32),
        scratch_types=[
            pltpu.VMEM((2, _R, _L), jnp.int32),
            pltpu.VMEM((2, chunk, d), jnp.float32),
            pltpu.SemaphoreType.DMA,
            pltpu.SemaphoreType.DMA,
        ],
    )
    def k(table_hbm, idx_hbm, out_hbm, idx_v, rows_v, sem0, sem1):
        sems = (sem0, sem1)
        wid = lax.axis_index("s") * _NC + lax.axis_index("c")
        row0 = wid * rows_w

        def start_chunk(g, s):
            r = row0 + g * _R
            pltpu.sync_copy(idx_hbm.at[pl.ds(r, _R)], idx_v.at[s])
            for j in range(_R):
                pltpu.make_async_copy(
                    table_hbm.at[idx_v.at[s].at[j]],
                    rows_v.at[s].at[pl.ds(j * _L, _L)],
                    sems[s],
                ).start()

        def finish_chunk(g, s):
            for j in range(_R):
                pltpu.make_async_copy(
                    table_hbm.at[idx_v.at[s].at[j]],
                    rows_v.at[s].at[pl.ds(j * _L, _L)],
                    sems[s],
                ).wait()
            o = (row0 + g * _R) * _L
            pltpu.sync_copy(rows_v.at[s], out_hbm.at[pl.ds(o, chunk)])

        start_chunk(0, 0)

        def body(kk, carry):
            g = kk * 2
            for b in range(2):
                start_chunk(g + b + 1, 1 - b)
                finish_chunk(g + b, b)
            return carry

        lax.fori_loop(0, (nchunk - 2) // 2, body, 0)

        start_chunk(nchunk - 1, 1)
        finish_chunk(nchunk - 2, 0)
        finish_chunk(nchunk - 1, 1)

    return k(table, idx2d)


def kernel(token_ids, embeddings):
    b, t = token_ids.shape
    v, d = embeddings.shape
    n = b * t
    assert n % (_NW * _R * _L) == 0
    idx2d = token_ids.reshape(n // _L, _L).astype(jnp.int32)
    out = _gather_impl(n // _L, d, embeddings, idx2d)
    return out.reshape(b, t, d)


# speedup vs baseline: 1.8552x; 1.8552x over previous
import functools

import jax
import jax.numpy as jnp
from jax import lax
from jax.experimental import pallas as pl
from jax.experimental.pallas import tpu as pltpu
from jax.experimental.pallas import tpu_sc as plsc

_INFO = plsc.get_sparse_core_info()
_NC = _INFO.num_cores
_NS = _INFO.num_subcores
_NW = _NC * _NS
_L = 128
_R = 4


def _gather_impl(n_idx_rows, d, table, idx2d):
    rows_w = n_idx_rows // _NW
    nchunk = rows_w // _R
    chunk = _R * _L
    assert n_idx_rows % _NW == 0 and rows_w % _R == 0
    assert nchunk >= 2 and nchunk % 2 == 0

    mesh = plsc.VectorSubcoreMesh(core_axis_name="c", subcore_axis_name="s")

    @functools.partial(
        pl.kernel,
        mesh=mesh,
        compiler_params=pltpu.CompilerParams(use_tc_tiling_on_sc=False),
        out_type=jax.ShapeDtypeStruct((n_idx_rows * _L, d), jnp.float32),
        scratch_types=[
            pltpu.VMEM((2, _R, _L), jnp.int32),
            pltpu.VMEM((2, chunk, d), jnp.float32),
            pltpu.SemaphoreType.DMA,
            pltpu.SemaphoreType.DMA,
        ],
    )
    def k(table_hbm, idx_hbm, out_hbm, idx_v, rows_v, sem0, sem1):
        sems = (sem0, sem1)
        wid = lax.axis_index("s") * _NC + lax.axis_index("c")
        row0 = wid * rows_w

        def start_chunk(g, s):
            r = row0 + g * _R
            pltpu.sync_copy(idx_hbm.at[pl.ds(r, _R)], idx_v.at[s])
            for j in range(_R):
                pltpu.make_async_copy(
                    table_hbm.at[idx_v.at[s].at[j]],
                    rows_v.at[s].at[pl.ds(j * _L, _L)],
                    sems[s],
                ).start()

        def finish_chunk(g, s):
            for j in range(_R):
                pltpu.make_async_copy(
                    table_hbm.at[idx_v.at[s].at[j]],
                    rows_v.at[s].at[pl.ds(j * _L, _L)],
                    sems[s],
                ).wait()
            o = (row0 + g * _R) * _L
            pltpu.sync_copy(rows_v.at[s], out_hbm.at[pl.ds(o, chunk)])

        start_chunk(0, 0)

        def body(kk, carry):
            g = kk * 2
            for b in range(2):
                start_chunk(g + b + 1, 1 - b)
                finish_chunk(g + b, b)
            return carry

        lax.fori_loop(0, (nchunk - 2) // 2, body, 0)

        start_chunk(nchunk - 1, 1)
        finish_chunk(nchunk - 2, 0)
        finish_chunk(nchunk - 1, 1)

    return k(table, idx2d)


def kernel(token_ids, embeddings):
    b, t = token_ids.shape
    v, d = embeddings.shape
    n = b * t
    assert n % (_NW * _R * _L) == 0
    idx2d = token_ids.reshape(n // _L, _L).astype(jnp.int32)
    out = _gather_impl(n // _L, d, embeddings, idx2d)
    return out.reshape(b, t, d)


# speedup vs baseline: 1.8632x; 1.0043x over previous
import functools

import jax
import jax.numpy as jnp
from jax import lax
from jax.experimental import pallas as pl
from jax.experimental.pallas import tpu as pltpu
from jax.experimental.pallas import tpu_sc as plsc

_INFO = plsc.get_sparse_core_info()
_NC = _INFO.num_cores
_NS = _INFO.num_subcores
_NW = _NC * _NS
_L = 128


def _gather_impl(t_steps, b, d, table, idx_tb):
    bw = b // _NW
    assert b % _NW == 0 and bw % _L == 0 and t_steps % 2 == 0
    nj = bw // _L

    mesh = plsc.VectorSubcoreMesh(core_axis_name="c", subcore_axis_name="s")

    @functools.partial(
        pl.kernel,
        mesh=mesh,
        compiler_params=pltpu.CompilerParams(use_tc_tiling_on_sc=False),
        out_type=jax.ShapeDtypeStruct((b, t_steps, d), jnp.float32),
        scratch_types=[
            pltpu.VMEM((2, bw), jnp.int32),
            pltpu.VMEM((2, bw, d), jnp.float32),
            pltpu.SemaphoreType.DMA,
            pltpu.SemaphoreType.DMA,
        ],
    )
    def k(table_hbm, idx_hbm, out_hbm, idx_v, rows_v, sem0, sem1):
        sems = (sem0, sem1)
        wid = lax.axis_index("s") * _NC + lax.axis_index("c")
        b0 = wid * bw

        def start_chunk(t, s):
            pltpu.sync_copy(idx_hbm.at[t, pl.ds(b0, bw)], idx_v.at[s])
            for j in range(nj):
                pltpu.make_async_copy(
                    table_hbm.at[idx_v.at[s].at[pl.ds(j * _L, _L)]],
                    rows_v.at[s].at[pl.ds(j * _L, _L)],
                    sems[s],
                ).start()

        def finish_chunk(t, s):
            for j in range(nj):
                pltpu.make_async_copy(
                    table_hbm.at[idx_v.at[s].at[pl.ds(j * _L, _L)]],
                    rows_v.at[s].at[pl.ds(j * _L, _L)],
                    sems[s],
                ).wait()
            pltpu.sync_copy(rows_v.at[s], out_hbm.at[pl.ds(b0, bw), t])

        start_chunk(0, 0)

        def body(kk, carry):
            t = kk * 2
            for x in range(2):
                start_chunk(t + x + 1, 1 - x)
                finish_chunk(t + x, x)
            return carry

        lax.fori_loop(0, (t_steps - 2) // 2, body, 0)

        start_chunk(t_steps - 1, 1)
        finish_chunk(t_steps - 2, 0)
        finish_chunk(t_steps - 1, 1)

    return k(table, idx_tb)


def kernel(token_ids, embeddings):
    b, t = token_ids.shape
    v, d = embeddings.shape
    idx_tb = token_ids.T.astype(jnp.int32)
    return _gather_impl(t, b, d, embeddings, idx_tb)
